# baseline (speedup 1.0000x reference)
import jax
import jax.numpy as jnp
from jax.experimental import pallas as pl
from jax.experimental.pallas import tpu as pltpu

B = 16
Z = 9
CH = 256
X = 32
Y = 32
CHANNELS = 86


def _pos_body(row_ref, col_ref, hei_ref, out_ref):
    z = pl.program_id(1)
    cw = col_ref[...]
    rw = row_ref[...]
    h = hei_ref[pl.ds(z, 1), :][0]
    a = jnp.broadcast_to(cw.T[:, None, :], (CHANNELS, X, Y))
    b = jnp.broadcast_to(rw.T[:, :, None], (CHANNELS, X, Y))
    c = jnp.broadcast_to(h[: CH - 2 * CHANNELS][:, None, None],
                         (CH - 2 * CHANNELS, X, Y))
    out_ref[0, 0] = jnp.concatenate([a, b, c], axis=0)


def kernel(tensor, row_w, col_w, hei_w):
    del tensor
    out = pl.pallas_call(
        _pos_body,
        grid=(B, Z),
        in_specs=[
            pl.BlockSpec((X, CHANNELS), lambda b, z: (0, 0)),
            pl.BlockSpec((Y, CHANNELS), lambda b, z: (0, 0)),
            pl.BlockSpec((Z, CHANNELS), lambda b, z: (0, 0)),
        ],
        out_specs=pl.BlockSpec((1, 1, CH, X, Y), lambda b, z: (b, z, 0, 0, 0)),
        out_shape=jax.ShapeDtypeStruct((B, Z, CH, X, Y), jnp.float32),
        compiler_params=pltpu.CompilerParams(
            dimension_semantics=("parallel", "parallel"),
        ),
    )(row_w, col_w, hei_w)
    return out


# speedup vs baseline: 1.1900x; 1.1900x over previous
import jax
import jax.numpy as jnp
from jax.experimental import pallas as pl
from jax.experimental.pallas import tpu as pltpu

B = 16
Z = 9
CH = 256
X = 32
Y = 32
CHANNELS = 86
C_REST = CH - 2 * CHANNELS


def _pos_body(row_ref, col_ref, hei_ref, out_ref):
    z = pl.program_id(0)
    cw = col_ref[...]
    rw = row_ref[...]
    h = hei_ref[pl.ds(z, 1), :][0]

    r = jax.lax.broadcasted_iota(jnp.int32, (X, X * Y), 0)
    q = jax.lax.broadcasted_iota(jnp.int32, (X, X * Y), 1)
    sel_j = (q % Y == r).astype(jnp.float32)
    sel_i = (q // Y == r).astype(jnp.float32)

    dn = (((0,), (0,)), ((), ()))
    a = jax.lax.dot_general(cw, sel_j, dn,
                            preferred_element_type=jnp.float32)
    b = jax.lax.dot_general(rw, sel_i, dn,
                            preferred_element_type=jnp.float32)
    c = jnp.broadcast_to(h[:C_REST][:, None], (C_REST, X * Y))
    chunk = jnp.concatenate([a, b, c], axis=0)
    for bb in range(B):
        out_ref[bb] = chunk


def kernel(tensor, row_w, col_w, hei_w):
    del tensor
    out = pl.pallas_call(
        _pos_body,
        grid=(Z,),
        in_specs=[
            pl.BlockSpec((X, CHANNELS), lambda z: (0, 0)),
            pl.BlockSpec((Y, CHANNELS), lambda z: (0, 0)),
            pl.BlockSpec((Z, CHANNELS), lambda z: (0, 0)),
        ],
        out_specs=pl.BlockSpec((B, CH, X * Y), lambda z: (0, z, 0)),
        out_shape=jax.ShapeDtypeStruct((B, Z * CH, X * Y), jnp.float32),
        compiler_params=pltpu.CompilerParams(
            dimension_semantics=("parallel",),
        ),
    )(row_w, col_w, hei_w)
    return out.reshape(B, Z, CH, X, Y)


# speedup vs baseline: 11.9426x; 10.0357x over previous
import jax
import jax.numpy as jnp
from jax.experimental import pallas as pl
from jax.experimental.pallas import tpu as pltpu

B = 16
Z = 9
CH = 256
X = 32
Y = 32
CHANNELS = 86


def _pos_body(col_ref, row_ref, hei_ref, out_ref, scratch, sem):
    z = pl.program_id(0)
    cw = col_ref[...]
    rw = row_ref[...]
    hz = hei_ref[pl.ds(z, 1), :]

    ci = jax.lax.broadcasted_iota(jnp.int32, (X, Y, CH), 2)
    a = jnp.broadcast_to(cw[None, :, :], (X, Y, CH))
    b = jnp.broadcast_to(rw[:, None, :], (X, Y, CH))
    c = jnp.broadcast_to(hz[0][None, None, :], (X, Y, CH))
    val = jnp.where(ci < CHANNELS, a, jnp.where(ci < 2 * CHANNELS, b, c))
    scratch[pl.ds(z, 1)] = val[None]

    def _copies(zz):
        return [
            pltpu.make_async_copy(
                scratch.at[pl.ds(zz, 1)],
                out_ref.at[bb, pl.ds(zz, 1)],
                sem,
            )
            for bb in range(B)
        ]

    for cp in _copies(z):
        cp.start()

    @pl.when(z > 0)
    def _():
        for cp in _copies(z - 1):
            cp.wait()

    @pl.when(z == Z - 1)
    def _():
        for cp in _copies(z):
            cp.wait()


def kernel(tensor, row_w, col_w, hei_w):
    del tensor
    cw256 = jnp.pad(col_w, ((0, 0), (0, CH - CHANNELS)))
    rw256 = jnp.pad(row_w, ((0, 0), (CHANNELS, CH - 2 * CHANNELS)))
    hei256 = jnp.pad(hei_w[:, : CH - 2 * CHANNELS], ((0, 0), (2 * CHANNELS, 0)))
    out = pl.pallas_call(
        _pos_body,
        grid=(Z,),
        in_specs=[
            pl.BlockSpec((Y, CH), lambda z: (0, 0)),
            pl.BlockSpec((X, CH), lambda z: (0, 0)),
            pl.BlockSpec((Z, CH), lambda z: (0, 0)),
        ],
        out_specs=pl.BlockSpec(memory_space=pl.ANY),
        out_shape=jax.ShapeDtypeStruct((B, Z, X, Y, CH), jnp.float32),
        scratch_shapes=[
            pltpu.VMEM((Z, X, Y, CH), jnp.float32),
            pltpu.SemaphoreType.DMA,
        ],
        compiler_params=pltpu.CompilerParams(
            dimension_semantics=("arbitrary",),
        ),
    )(cw256, rw256, hei256)
    return jnp.transpose(out, (0, 1, 4, 2, 3))
